# Initial kernel scaffold; baseline (speedup 1.0000x reference)
import jax
import jax.numpy as jnp
from jax.experimental import pallas as pl


def kernel(x, table):
    raise NotImplementedError("write your pallas kernel here")


# speedup vs baseline: 3.3090x; 3.3090x over previous
import jax
import jax.numpy as jnp
from jax import lax
from jax.experimental import pallas as pl
from jax.experimental.pallas import tpu as pltpu
from jax.experimental.pallas import tpu_sc as plsc

NC, NS = 2, 16
NW = NC * NS
C = 128
NBUF = 5


def _gather_body(table_hbm, idx_hbm, out_hbm, idx_v, bufs, gsem, osem):
    nchunk = idx_hbm.shape[1]
    ngroup = nchunk // NBUF
    wid = lax.axis_index("s") * NC + lax.axis_index("c")
    base = wid * (nchunk * C)

    pltpu.sync_copy(idx_hbm.at[wid], idx_v)

    def start_gather(j, b):
        return pltpu.async_copy(table_hbm.at[idx_v.at[j]], bufs.at[b], gsem.at[b])

    def start_out(j, b):
        pltpu.async_copy(bufs.at[b], out_hbm.at[pl.ds(base + j * C, C)], osem.at[b])

    def wait_out(b):
        pltpu.make_async_copy(bufs.at[b], out_hbm.at[pl.ds(base, C)], osem.at[b]).wait()

    hs = [start_gather(b, b) for b in range(NBUF)]
    for b in range(NBUF):
        hs[b].wait()
        start_out(b, b)

    def group(g, carry):
        hg = []
        for b in range(NBUF):
            wait_out(b)
            hg.append(start_gather(g * NBUF + b, b))
        for b in range(NBUF):
            hg[b].wait()
            start_out(g * NBUF + b, b)
        return carry

    lax.fori_loop(1, ngroup, group, 0)

    for b in range(NBUF):
        wait_out(b)


def kernel(x, table):
    B, S = x.shape
    V, D = table.shape
    N = B * S
    rows_w = N // NW
    nchunk = rows_w // C
    idx3 = x.reshape(NW, nchunk, C).astype(jnp.int32)
    mesh = plsc.VectorSubcoreMesh(core_axis_name="c", subcore_axis_name="s")
    out = pl.kernel(
        _gather_body,
        out_type=jax.ShapeDtypeStruct((N, D), table.dtype),
        mesh=mesh,
        scratch_types=[
            pltpu.VMEM((nchunk, C), jnp.int32),
            pltpu.VMEM((NBUF, C, D), jnp.float32),
            pltpu.SemaphoreType.DMA((NBUF,)),
            pltpu.SemaphoreType.DMA((NBUF,)),
        ],
    )(table, idx3)
    return out.reshape(B, S, D)


# speedup vs baseline: 5.9136x; 1.7871x over previous
import jax
import jax.numpy as jnp
from jax import lax
from jax.experimental import pallas as pl
from jax.experimental.pallas import tpu as pltpu
from jax.experimental.pallas import tpu_sc as plsc

NC, NS = 2, 16
NW = NC * NS
NBUF = 8


def _gather_body(table_hbm, x_hbm, out_hbm, idx_v, bufs, gsem, osem):
    rows_w = x_hbm.shape[0] // NW
    ngroup = rows_w // NBUF
    wid = lax.axis_index("s") * NC + lax.axis_index("c")
    base = wid * rows_w

    pltpu.sync_copy(x_hbm.at[pl.ds(base, rows_w)], idx_v)

    def start_gather(j, b):
        return pltpu.async_copy(table_hbm.at[idx_v.at[j]], bufs.at[b], gsem.at[b])

    def start_out(j, b):
        pltpu.async_copy(bufs.at[b], out_hbm.at[base + j], osem.at[b])

    def wait_out(b):
        pltpu.make_async_copy(bufs.at[b], out_hbm.at[base], osem.at[b]).wait()

    hs = [start_gather(b, b) for b in range(NBUF)]
    for b in range(NBUF):
        hs[b].wait()
        start_out(b, b)

    def group(g, carry):
        hg = []
        for b in range(NBUF):
            wait_out(b)
            hg.append(start_gather(g * NBUF + b, b))
        for b in range(NBUF):
            hg[b].wait()
            start_out(g * NBUF + b, b)
        return carry

    lax.fori_loop(1, ngroup, group, 0)

    for b in range(NBUF):
        wait_out(b)


def kernel(x, table):
    B, S = x.shape
    V, D = table.shape
    rows_w = B // NW
    mesh = plsc.VectorSubcoreMesh(core_axis_name="c", subcore_axis_name="s")
    out = pl.kernel(
        _gather_body,
        out_type=jax.ShapeDtypeStruct((B, S, D), table.dtype),
        mesh=mesh,
        scratch_types=[
            pltpu.VMEM((rows_w, S), jnp.int32),
            pltpu.VMEM((NBUF, S, D), jnp.float32),
            pltpu.SemaphoreType.DMA((NBUF,)),
            pltpu.SemaphoreType.DMA((NBUF,)),
        ],
    )(table, x.astype(jnp.int32))
    return out
